# baseline (speedup 1.0000x reference)
import jax
import jax.numpy as jnp
from jax.experimental import pallas as pl
from jax.experimental.pallas import tpu as pltpu

_NUM_BINS = 100000
_ROWS, _COLS = 16384, 100
_BC = 1024
_GRID = _ROWS // _BC


def _hash_mod(x):
    c = jnp.uint32(0x45D9F3B)
    x = (x ^ (x >> 16)) * c
    x = (x ^ (x >> 16)) * c
    h = x ^ (x >> 16)
    qf = (h >> 1).astype(jnp.int32).astype(jnp.float32) * jnp.float32(1.9999923e-05)
    ri = h.astype(jnp.int32) - qf.astype(jnp.int32) * jnp.int32(_NUM_BINS)
    t = ri - jnp.int32(_NUM_BINS)
    return jnp.where(t >= 0, t, ri)


def _body(x_ref, o_ref):
    o_ref[...] = _hash_mod(x_ref[...].astype(jnp.uint32))


def _tc_hash_t(xt):
    spec = pl.BlockSpec((_COLS, _BC), lambda i: (0, i))
    return pl.pallas_call(
        _body,
        grid=(_GRID,),
        in_specs=[spec],
        out_specs=spec,
        out_shape=jax.ShapeDtypeStruct((_COLS, _ROWS), jnp.int32),
        compiler_params=pltpu.CompilerParams(
            dimension_semantics=("parallel",)),
    )(xt)


def kernel(inputs):
    return _tc_hash_t(inputs.T).T


# speedup vs baseline: 1.3247x; 1.3247x over previous
import jax
import jax.numpy as jnp
from jax.experimental import pallas as pl
from jax.experimental.pallas import tpu as pltpu

_NUM_BINS = 100000
_ROWS, _COLS = 16384, 100
_BC = 2048
_GRID = _ROWS // _BC


def _hash_mod(x):
    c = jnp.uint32(0x45D9F3B)
    x = (x ^ (x >> 16)) * c
    x = (x ^ (x >> 16)) * c
    h = x ^ (x >> 16)
    qf = (h >> 1).astype(jnp.int32).astype(jnp.float32) * jnp.float32(1.9999923e-05)
    ri = h.astype(jnp.int32) - qf.astype(jnp.int32) * jnp.int32(_NUM_BINS)
    t = ri - jnp.int32(_NUM_BINS)
    return jnp.where(t >= 0, t, ri)


def _body(x_ref, o_ref):
    o_ref[...] = _hash_mod(x_ref[...].astype(jnp.uint32))


def _tc_hash_t(xt):
    spec = pl.BlockSpec((_COLS, _BC), lambda i: (0, i))
    return pl.pallas_call(
        _body,
        grid=(_GRID,),
        in_specs=[spec],
        out_specs=spec,
        out_shape=jax.ShapeDtypeStruct((_COLS, _ROWS), jnp.int32),
        compiler_params=pltpu.CompilerParams(
            dimension_semantics=("parallel",)),
    )(xt)


def kernel(inputs):
    return _tc_hash_t(inputs.T).T


# speedup vs baseline: 1.3396x; 1.0112x over previous
import jax
import jax.numpy as jnp
from jax.experimental import pallas as pl
from jax.experimental.pallas import tpu as pltpu

_NUM_BINS = 100000
_ROWS, _COLS = 16384, 100
_BC = 2048
_GRID = _ROWS // _BC


def _hash_mod(x):
    c = jnp.uint32(0x45D9F3B)
    x = (x ^ (x >> 16)) * c
    x = (x ^ (x >> 16)) * c
    h = x ^ (x >> 16)
    qf = (h >> 1).astype(jnp.int32).astype(jnp.float32) * jnp.float32(1.9999923e-05)
    ri = h.astype(jnp.int32) - qf.astype(jnp.int32) * jnp.int32(_NUM_BINS)
    t = ri - jnp.int32(_NUM_BINS)
    return jnp.where(t >= 0, t, ri)


def _body(x_ref, o_ref):
    o_ref[...] = _hash_mod(x_ref[...].astype(jnp.uint32))


def _tc_hash_t(xt):
    spec = pl.BlockSpec((_COLS, _BC), lambda i: (0, i))
    return pl.pallas_call(
        _body,
        grid=(_GRID,),
        in_specs=[spec],
        out_specs=spec,
        out_shape=jax.ShapeDtypeStruct((_COLS, _ROWS), jnp.int32),
        compiler_params=pltpu.CompilerParams(
            dimension_semantics=("arbitrary",)),
    )(xt)


def kernel(inputs):
    return _tc_hash_t(inputs.T).T


# speedup vs baseline: 1.4317x; 1.0687x over previous
import jax
import jax.numpy as jnp
from jax.experimental import pallas as pl
from jax.experimental.pallas import tpu as pltpu

_NUM_BINS = 100000
_ROWS, _COLS = 16384, 100
_BC = 2048
_NC = _ROWS // _BC
_NBUF = 2


def _hash_mod(x):
    c = jnp.uint32(0x45D9F3B)
    x = (x ^ (x >> 16)) * c
    x = (x ^ (x >> 16)) * c
    h = x ^ (x >> 16)
    qf = (h >> 1).astype(jnp.int32).astype(jnp.float32) * jnp.float32(1.9999923e-05)
    ri = h.astype(jnp.int32) - qf.astype(jnp.int32) * jnp.int32(_NUM_BINS)
    t = ri - jnp.int32(_NUM_BINS)
    return jnp.where(t >= 0, t, ri)


def _body(x_hbm, o_hbm, in_v, out_v, in_sem, out_sem):
    def copy_in(i):
        return pltpu.make_async_copy(
            x_hbm.at[:, pl.ds(i * _BC, _BC)], in_v.at[i % _NBUF],
            in_sem.at[i % _NBUF])

    def copy_out(i):
        return pltpu.make_async_copy(
            out_v.at[i % _NBUF], o_hbm.at[:, pl.ds(i * _BC, _BC)],
            out_sem.at[i % _NBUF])

    copy_in(0).start()
    for i in range(_NC):
        if i + 1 < _NC:
            copy_in(i + 1).start()
        copy_in(i).wait()
        if i >= _NBUF:
            copy_out(i - _NBUF).wait()
        out_v[i % _NBUF] = _hash_mod(in_v[i % _NBUF].astype(jnp.uint32))
        copy_out(i).start()
    for i in range(max(_NC - _NBUF, 0), _NC):
        copy_out(i).wait()


def _tc_hash_t(xt):
    return pl.pallas_call(
        _body,
        in_specs=[pl.BlockSpec(memory_space=pltpu.HBM)],
        out_specs=pl.BlockSpec(memory_space=pltpu.HBM),
        out_shape=jax.ShapeDtypeStruct((_COLS, _ROWS), jnp.int32),
        scratch_shapes=[
            pltpu.VMEM((_NBUF, _COLS, _BC), jnp.int32),
            pltpu.VMEM((_NBUF, _COLS, _BC), jnp.int32),
            pltpu.SemaphoreType.DMA((_NBUF,)),
            pltpu.SemaphoreType.DMA((_NBUF,)),
        ],
    )(xt)


def kernel(inputs):
    return _tc_hash_t(inputs.T).T


# speedup vs baseline: 1.4386x; 1.0048x over previous
import jax
import jax.numpy as jnp
from jax.experimental import pallas as pl
from jax.experimental.pallas import tpu as pltpu

_NUM_BINS = 100000
_ROWS, _COLS = 16384, 100
_BC = 1024
_NC = _ROWS // _BC
_NBUF = 3


def _hash_mod(x):
    c = jnp.uint32(0x45D9F3B)
    x = (x ^ (x >> 16)) * c
    x = (x ^ (x >> 16)) * c
    h = x ^ (x >> 16)
    qf = (h >> 1).astype(jnp.int32).astype(jnp.float32) * jnp.float32(1.9999923e-05)
    ri = h.astype(jnp.int32) - qf.astype(jnp.int32) * jnp.int32(_NUM_BINS)
    t = ri - jnp.int32(_NUM_BINS)
    return jnp.where(t >= 0, t, ri)


def _body(x_hbm, o_hbm, in_v, out_v, in_sem, out_sem):
    def copy_in(i):
        return pltpu.make_async_copy(
            x_hbm.at[:, pl.ds(i * _BC, _BC)], in_v.at[i % _NBUF],
            in_sem.at[i % _NBUF])

    def copy_out(i):
        return pltpu.make_async_copy(
            out_v.at[i % _NBUF], o_hbm.at[:, pl.ds(i * _BC, _BC)],
            out_sem.at[i % _NBUF])

    for j in range(min(_NBUF, _NC)):
        copy_in(j).start()
    for i in range(_NC):
        copy_in(i).wait()
        if i >= _NBUF:
            copy_out(i - _NBUF).wait()
        out_v[i % _NBUF] = _hash_mod(in_v[i % _NBUF].astype(jnp.uint32))
        copy_out(i).start()
        if i + _NBUF < _NC:
            copy_in(i + _NBUF).start()
    for i in range(max(_NC - _NBUF, 0), _NC):
        copy_out(i).wait()


def _tc_hash_t(xt):
    return pl.pallas_call(
        _body,
        in_specs=[pl.BlockSpec(memory_space=pltpu.HBM)],
        out_specs=pl.BlockSpec(memory_space=pltpu.HBM),
        out_shape=jax.ShapeDtypeStruct((_COLS, _ROWS), jnp.int32),
        scratch_shapes=[
            pltpu.VMEM((_NBUF, _COLS, _BC), jnp.int32),
            pltpu.VMEM((_NBUF, _COLS, _BC), jnp.int32),
            pltpu.SemaphoreType.DMA((_NBUF,)),
            pltpu.SemaphoreType.DMA((_NBUF,)),
        ],
    )(xt)


def kernel(inputs):
    return _tc_hash_t(inputs.T).T


# speedup vs baseline: 1.6129x; 1.1212x over previous
import jax
import jax.numpy as jnp
from jax.experimental import pallas as pl
from jax.experimental.pallas import tpu as pltpu

_NUM_BINS = 100000
_ROWS, _COLS = 16384, 100
_BH = 8
_CHUNKS = [(i * _BH, min(_BH, _COLS - i * _BH))
           for i in range((_COLS + _BH - 1) // _BH)]
_NBUF = 3


def _hash_mod(x):
    c = jnp.uint32(0x45D9F3B)
    x = (x ^ (x >> 16)) * c
    x = (x ^ (x >> 16)) * c
    h = x ^ (x >> 16)
    qf = (h >> 1).astype(jnp.int32).astype(jnp.float32) * jnp.float32(1.9999923e-05)
    ri = h.astype(jnp.int32) - qf.astype(jnp.int32) * jnp.int32(_NUM_BINS)
    t = ri - jnp.int32(_NUM_BINS)
    return jnp.where(t >= 0, t, ri)


def _body(x_hbm, o_hbm, in_v, out_v, in_sem, out_sem):
    def copy_in(i):
        r, h = _CHUNKS[i]
        return pltpu.make_async_copy(
            x_hbm.at[pl.ds(r, h), :], in_v.at[i % _NBUF, pl.ds(0, h)],
            in_sem.at[i % _NBUF])

    def copy_out(i):
        r, h = _CHUNKS[i]
        return pltpu.make_async_copy(
            out_v.at[i % _NBUF, pl.ds(0, h)], o_hbm.at[pl.ds(r, h), :],
            out_sem.at[i % _NBUF])

    n = len(_CHUNKS)
    for j in range(min(_NBUF, n)):
        copy_in(j).start()
    for i in range(n):
        _, h = _CHUNKS[i]
        copy_in(i).wait()
        if i >= _NBUF:
            copy_out(i - _NBUF).wait()
        out_v[i % _NBUF, pl.ds(0, h)] = _hash_mod(
            in_v[i % _NBUF, pl.ds(0, h)].astype(jnp.uint32))
        copy_out(i).start()
        if i + _NBUF < n:
            copy_in(i + _NBUF).start()
    for i in range(max(n - _NBUF, 0), n):
        copy_out(i).wait()


def _tc_hash_t(xt):
    return pl.pallas_call(
        _body,
        in_specs=[pl.BlockSpec(memory_space=pltpu.HBM)],
        out_specs=pl.BlockSpec(memory_space=pltpu.HBM),
        out_shape=jax.ShapeDtypeStruct((_COLS, _ROWS), jnp.int32),
        scratch_shapes=[
            pltpu.VMEM((_NBUF, _BH, _ROWS), jnp.int32),
            pltpu.VMEM((_NBUF, _BH, _ROWS), jnp.int32),
            pltpu.SemaphoreType.DMA((_NBUF,)),
            pltpu.SemaphoreType.DMA((_NBUF,)),
        ],
    )(xt)


def kernel(inputs):
    return _tc_hash_t(inputs.T).T


# speedup vs baseline: 2.0442x; 1.2674x over previous
import jax
import jax.numpy as jnp
from jax.experimental import pallas as pl
from jax.experimental.pallas import tpu as pltpu

_NUM_BINS = 100000
_ROWS, _COLS = 16384, 100
_BH = 8
_CHUNKS = [(i * _BH, min(_BH, _COLS - i * _BH))
           for i in range((_COLS + _BH - 1) // _BH)]
_NBUF = 5


def _hash_mod(x):
    c = jnp.uint32(0x45D9F3B)
    x = (x ^ (x >> 16)) * c
    x = (x ^ (x >> 16)) * c
    h = x ^ (x >> 16)
    qf = (h >> 1).astype(jnp.int32).astype(jnp.float32) * jnp.float32(1.9999923e-05)
    ri = h.astype(jnp.int32) - qf.astype(jnp.int32) * jnp.int32(_NUM_BINS)
    t = ri - jnp.int32(_NUM_BINS)
    return jnp.where(t >= 0, t, ri)


def _body(x_hbm, o_hbm, in_v, out_v, in_sem, out_sem):
    def copy_in(i):
        r, h = _CHUNKS[i]
        return pltpu.make_async_copy(
            x_hbm.at[pl.ds(r, h), :], in_v.at[i % _NBUF, pl.ds(0, h)],
            in_sem.at[i % _NBUF])

    def copy_out(i):
        r, h = _CHUNKS[i]
        return pltpu.make_async_copy(
            out_v.at[i % _NBUF, pl.ds(0, h)], o_hbm.at[pl.ds(r, h), :],
            out_sem.at[i % _NBUF])

    n = len(_CHUNKS)
    for j in range(min(_NBUF, n)):
        copy_in(j).start()
    for i in range(n):
        _, h = _CHUNKS[i]
        copy_in(i).wait()
        if i >= _NBUF:
            copy_out(i - _NBUF).wait()
        out_v[i % _NBUF, pl.ds(0, h)] = _hash_mod(
            in_v[i % _NBUF, pl.ds(0, h)].astype(jnp.uint32))
        copy_out(i).start()
        if i + _NBUF < n:
            copy_in(i + _NBUF).start()
    for i in range(max(n - _NBUF, 0), n):
        copy_out(i).wait()


def _tc_hash_t(xt):
    return pl.pallas_call(
        _body,
        in_specs=[pl.BlockSpec(memory_space=pltpu.HBM)],
        out_specs=pl.BlockSpec(memory_space=pltpu.HBM),
        out_shape=jax.ShapeDtypeStruct((_COLS, _ROWS), jnp.int32),
        scratch_shapes=[
            pltpu.VMEM((_NBUF, _BH, _ROWS), jnp.int32),
            pltpu.VMEM((_NBUF, _BH, _ROWS), jnp.int32),
            pltpu.SemaphoreType.DMA((_NBUF,)),
            pltpu.SemaphoreType.DMA((_NBUF,)),
        ],
    )(xt)


def kernel(inputs):
    return _tc_hash_t(inputs.T).T


# speedup vs baseline: 2.1727x; 1.0629x over previous
import jax
import jax.numpy as jnp
from jax.experimental import pallas as pl
from jax.experimental.pallas import tpu as pltpu

_NUM_BINS = 100000
_ROWS, _COLS = 16384, 100
_BH = 8
_CHUNKS = [(i * _BH, min(_BH, _COLS - i * _BH))
           for i in range((_COLS + _BH - 1) // _BH)]
_NBUF = 7


def _hash_mod(x):
    c = jnp.uint32(0x45D9F3B)
    x = (x ^ (x >> 16)) * c
    x = (x ^ (x >> 16)) * c
    h = x ^ (x >> 16)
    qf = (h >> 1).astype(jnp.int32).astype(jnp.float32) * jnp.float32(1.9999923e-05)
    ri = h.astype(jnp.int32) - qf.astype(jnp.int32) * jnp.int32(_NUM_BINS)
    t = ri - jnp.int32(_NUM_BINS)
    return jnp.where(t >= 0, t, ri)


def _body(x_hbm, o_hbm, in_v, out_v, in_sem, out_sem):
    def copy_in(i):
        r, h = _CHUNKS[i]
        return pltpu.make_async_copy(
            x_hbm.at[pl.ds(r, h), :], in_v.at[i % _NBUF, pl.ds(0, h)],
            in_sem.at[i % _NBUF])

    def copy_out(i):
        r, h = _CHUNKS[i]
        return pltpu.make_async_copy(
            out_v.at[i % _NBUF, pl.ds(0, h)], o_hbm.at[pl.ds(r, h), :],
            out_sem.at[i % _NBUF])

    n = len(_CHUNKS)
    for j in range(min(_NBUF, n)):
        copy_in(j).start()
    for i in range(n):
        _, h = _CHUNKS[i]
        copy_in(i).wait()
        if i >= _NBUF:
            copy_out(i - _NBUF).wait()
        out_v[i % _NBUF, pl.ds(0, h)] = _hash_mod(
            in_v[i % _NBUF, pl.ds(0, h)].astype(jnp.uint32))
        copy_out(i).start()
        if i + _NBUF < n:
            copy_in(i + _NBUF).start()
    for i in range(max(n - _NBUF, 0), n):
        copy_out(i).wait()


def _tc_hash_t(xt):
    return pl.pallas_call(
        _body,
        in_specs=[pl.BlockSpec(memory_space=pltpu.HBM)],
        out_specs=pl.BlockSpec(memory_space=pltpu.HBM),
        out_shape=jax.ShapeDtypeStruct((_COLS, _ROWS), jnp.int32),
        scratch_shapes=[
            pltpu.VMEM((_NBUF, _BH, _ROWS), jnp.int32),
            pltpu.VMEM((_NBUF, _BH, _ROWS), jnp.int32),
            pltpu.SemaphoreType.DMA((_NBUF,)),
            pltpu.SemaphoreType.DMA((_NBUF,)),
        ],
    )(xt)


def kernel(inputs):
    return _tc_hash_t(inputs.T).T
